# baseline (speedup 1.0000x reference)
import functools

import jax
import jax.numpy as jnp
from jax import lax
from jax.experimental import pallas as pl
from jax.experimental.pallas import tpu as pltpu
from jax.experimental.pallas import tpu_sc as plsc

N = 32768
E = 64
D = 256
CAP = 640
ECAP = E * CAP
TB = 512
NBLK = N // TB
XYZ_FREQ = 12
DIR_FREQ = 4

SC_NC = 2
SC_NS = 16
NW = SC_NC * SC_NS
TOK_PER_W = N // NW
CHUNK = 128
NCHUNK = TOK_PER_W // CHUNK


def _posenc_cols(v, n_freqs):
    cols = [v]
    for f in range(n_freqs):
        s = float(2.0 ** f)
        cols.append(jnp.sin(v * s))
        cols.append(jnp.cos(v * s))
    return jnp.concatenate(cols, axis=-1)


def _k1_body(x_ref, wxyz_ref, bxyz_ref, weg1_ref, beg1_ref, weg2_ref,
             beg2_ref, lng_ref, lnb_ref, wg_ref,
             h_ref, loc_ref, cloc_ref, score_ref, cnt_ref):
    @pl.when(pl.program_id(0) == 0)
    def _():
        cnt_ref[...] = jnp.zeros((1, E), jnp.float32)

    xb = x_ref[...]
    xyz = xb[:, 0:3]
    pe = _posenc_cols(xyz, XYZ_FREQ)
    h = jnp.dot(pe, wxyz_ref[...], preferred_element_type=jnp.float32)
    h = h + bxyz_ref[...]
    h_ref[...] = h

    eg = jnp.maximum(
        jnp.dot(h, weg1_ref[...], preferred_element_type=jnp.float32)
        + beg1_ref[...], 0.0)
    eg = jnp.dot(eg, weg2_ref[...], preferred_element_type=jnp.float32)
    eg = eg + beg2_ref[...]
    m = jnp.mean(eg, axis=-1, keepdims=True)
    v = jnp.mean((eg - m) ** 2, axis=-1, keepdims=True)
    gi = (eg - m) / jnp.sqrt(v + 1e-5) * lng_ref[...] + lnb_ref[...]

    logits = jnp.dot(gi, wg_ref[...], preferred_element_type=jnp.float32)
    lmax = jnp.max(logits, axis=-1, keepdims=True)
    score = 1.0 / jnp.sum(jnp.exp(logits - lmax), axis=-1, keepdims=True)

    eq = (logits == lmax).astype(jnp.float32)
    utri = (lax.broadcasted_iota(jnp.int32, (E, E), 0)
            <= lax.broadcasted_iota(jnp.int32, (E, E), 1)).astype(jnp.float32)
    csum1 = jnp.dot(eq, utri, preferred_element_type=jnp.float32)
    oh = eq * (csum1 == 1.0).astype(jnp.float32)

    ltri = (lax.broadcasted_iota(jnp.int32, (TB, TB), 0)
            >= lax.broadcasted_iota(jnp.int32, (TB, TB), 1)).astype(jnp.float32)
    csum0 = jnp.dot(ltri, oh, preferred_element_type=jnp.float32)
    pos_in = jnp.sum(csum0 * oh, axis=-1, keepdims=True) - 1.0

    cnt0 = cnt_ref[...]
    prev = lax.dot_general(oh, cnt0, (((1,), (1,)), ((), ())),
                           preferred_element_type=jnp.float32)
    pos = pos_in + prev
    cnt_ref[...] = cnt0 + jnp.sum(oh, axis=0, keepdims=True)

    col = lax.broadcasted_iota(jnp.int32, (TB, E), 1).astype(jnp.float32)
    idx_f = jnp.sum(oh * col, axis=-1, keepdims=True)
    keep = pos < float(CAP)
    loc_f = jnp.where(keep, idx_f * float(CAP) + pos, float(ECAP))
    loc = loc_f.astype(jnp.int32)
    loc_ref[...] = loc
    cloc_ref[...] = jnp.minimum(loc, ECAP - 1)
    score_ref[...] = jnp.where(keep, score, 0.0)


def _run_k1(x, W_xyz, b_xyz, W_eg1, b_eg1, W_eg2, b_eg2, ln_g, ln_b, Wg):
    full = lambda shape: pl.BlockSpec(shape, lambda i: (0,) * len(shape))
    return pl.pallas_call(
        _k1_body,
        grid=(NBLK,),
        in_specs=[
            pl.BlockSpec((TB, 7), lambda i: (i, 0)),
            full((75, D)), full((D,)), full((D, D)), full((D,)),
            full((D, D)), full((D,)), full((D,)), full((D,)), full((D, E)),
        ],
        out_specs=[
            pl.BlockSpec((TB, D), lambda i: (i, 0)),
            pl.BlockSpec((TB, 1), lambda i: (i, 0)),
            pl.BlockSpec((TB, 1), lambda i: (i, 0)),
            pl.BlockSpec((TB, 1), lambda i: (i, 0)),
        ],
        out_shape=[
            jax.ShapeDtypeStruct((N, D), jnp.float32),
            jax.ShapeDtypeStruct((N, 1), jnp.int32),
            jax.ShapeDtypeStruct((N, 1), jnp.int32),
            jax.ShapeDtypeStruct((N, 1), jnp.float32),
        ],
        scratch_shapes=[pltpu.VMEM((1, E), jnp.float32)],
        compiler_params=pltpu.CompilerParams(
            dimension_semantics=("arbitrary",)),
    )(x, W_xyz, b_xyz, W_eg1, b_eg1, W_eg2, b_eg2, ln_g, ln_b, Wg)


NBUF = 3


def _k2_body(h_hbm, loc_hbm, buf_hbm, idx_v, b0, b1, b2, lsem, ssem):
    wid = lax.axis_index("s") * SC_NC + lax.axis_index("c")
    bufs = [b0, b1, b2]
    pltpu.sync_copy(loc_hbm.at[pl.ds(wid * NCHUNK, NCHUNK)], idx_v)
    loads, scats = {}, {}
    for c in range(NBUF):
        base = wid * TOK_PER_W + c * CHUNK
        loads[c] = pltpu.async_copy(
            h_hbm.at[pl.ds(base, CHUNK)], bufs[c], lsem)
    for c in range(NCHUNK):
        if c >= NBUF:
            scats[c - NBUF].wait()
            base = wid * TOK_PER_W + c * CHUNK
            loads[c] = pltpu.async_copy(
                h_hbm.at[pl.ds(base, CHUNK)], bufs[c % NBUF], lsem)
        loads[c].wait()
        scats[c] = pltpu.async_copy(
            bufs[c % NBUF], buf_hbm.at[idx_v.at[c]], ssem)
    for c in range(NCHUNK - NBUF, NCHUNK):
        scats[c].wait()


def _run_k2(h, loc):
    mesh = plsc.VectorSubcoreMesh(core_axis_name="c", subcore_axis_name="s",
                                  num_cores=SC_NC, num_subcores=SC_NS)
    f = functools.partial(
        pl.kernel, mesh=mesh,
        out_type=jax.ShapeDtypeStruct((ECAP + 8, 2, 128), jnp.float32),
        scratch_types=[
            pltpu.VMEM((NCHUNK, CHUNK), jnp.int32),
            pltpu.VMEM((CHUNK, 2, 128), jnp.float32),
            pltpu.VMEM((CHUNK, 2, 128), jnp.float32),
            pltpu.VMEM((CHUNK, 2, 128), jnp.float32),
            pltpu.SemaphoreType.DMA,
            pltpu.SemaphoreType.DMA,
        ],
    )(_k2_body)
    return f(h.reshape(N, 2, 128), loc.reshape(NW * NCHUNK, CHUNK))


def _k3_body(xin_ref, wpre_ref, bpre_ref, wskip_ref, bskip_ref,
             wpost_ref, bpost_ref, out_ref):
    xin = xin_ref[...]
    hb = xin
    for j in range(3):
        hb = jnp.maximum(
            jnp.dot(hb, wpre_ref[0, j], preferred_element_type=jnp.float32)
            + bpre_ref[0, j], 0.0)
    hb2 = jnp.concatenate([xin, hb], axis=-1)
    hb = jnp.maximum(
        jnp.dot(hb2, wskip_ref[0], preferred_element_type=jnp.float32)
        + bskip_ref[0, 0], 0.0)
    for j in range(3):
        hb = jnp.maximum(
            jnp.dot(hb, wpost_ref[0, j], preferred_element_type=jnp.float32)
            + bpost_ref[0, j], 0.0)
    out_ref[...] = hb


def _run_k3(buf, We_pre, be_pre, We_skip, be_skip, We_post, be_post):
    return pl.pallas_call(
        _k3_body,
        grid=(E,),
        in_specs=[
            pl.BlockSpec((CAP, D), lambda e: (e, 0)),
            pl.BlockSpec((1, 3, D, D), lambda e: (e, 0, 0, 0)),
            pl.BlockSpec((1, 3, D), lambda e: (e, 0, 0)),
            pl.BlockSpec((1, 2 * D, D), lambda e: (e, 0, 0)),
            pl.BlockSpec((1, 1, D), lambda e: (e, 0, 0)),
            pl.BlockSpec((1, 3, D, D), lambda e: (e, 0, 0, 0)),
            pl.BlockSpec((1, 3, D), lambda e: (e, 0, 0)),
        ],
        out_specs=pl.BlockSpec((CAP, D), lambda e: (e, 0)),
        out_shape=jax.ShapeDtypeStruct((ECAP, D), jnp.float32),
        compiler_params=pltpu.CompilerParams(
            dimension_semantics=("arbitrary",)),
    )(buf, We_pre, be_pre, We_skip, be_skip.reshape(E, 1, D),
      We_post, be_post)


def _k4_body(flat_hbm, cloc_hbm, y_hbm, idx_v, b0, b1, b2, gsem, ssem):
    wid = lax.axis_index("s") * SC_NC + lax.axis_index("c")
    bufs = [b0, b1, b2]
    pltpu.sync_copy(cloc_hbm.at[pl.ds(wid * NCHUNK, NCHUNK)], idx_v)
    gaths, stores = {}, {}
    for c in range(NBUF):
        gaths[c] = pltpu.async_copy(
            flat_hbm.at[idx_v.at[c]], bufs[c], gsem)
    for c in range(NCHUNK):
        if c >= NBUF:
            stores[c - NBUF].wait()
            gaths[c] = pltpu.async_copy(
                flat_hbm.at[idx_v.at[c]], bufs[c % NBUF], gsem)
        gaths[c].wait()
        base = wid * TOK_PER_W + c * CHUNK
        stores[c] = pltpu.async_copy(
            bufs[c % NBUF], y_hbm.at[pl.ds(base, CHUNK)], ssem)
    for c in range(NCHUNK - NBUF, NCHUNK):
        stores[c].wait()


def _run_k4(flat, cloc):
    mesh = plsc.VectorSubcoreMesh(core_axis_name="c", subcore_axis_name="s",
                                  num_cores=SC_NC, num_subcores=SC_NS)
    f = functools.partial(
        pl.kernel, mesh=mesh,
        out_type=jax.ShapeDtypeStruct((N, 2, 128), jnp.float32),
        scratch_types=[
            pltpu.VMEM((NCHUNK, CHUNK), jnp.int32),
            pltpu.VMEM((CHUNK, 2, 128), jnp.float32),
            pltpu.VMEM((CHUNK, 2, 128), jnp.float32),
            pltpu.VMEM((CHUNK, 2, 128), jnp.float32),
            pltpu.SemaphoreType.DMA,
            pltpu.SemaphoreType.DMA,
        ],
    )(_k4_body)
    return f(flat.reshape(ECAP, 2, 128), cloc.reshape(NW * NCHUNK, CHUNK))


def _k4b_body(emb_hbm, aidx_hbm, app_hbm, aidx_v, b0, b1, b2, gsem, ssem):
    wid = lax.axis_index("s") * SC_NC + lax.axis_index("c")
    bufs = [b0, b1, b2]
    pltpu.sync_copy(aidx_hbm.at[pl.ds(wid * NCHUNK, NCHUNK)], aidx_v)
    gaths, stores = {}, {}
    for c in range(NBUF):
        gaths[c] = pltpu.async_copy(
            emb_hbm.at[aidx_v.at[c]], bufs[c], gsem)
    for c in range(NCHUNK):
        if c >= NBUF:
            stores[c - NBUF].wait()
            gaths[c] = pltpu.async_copy(
                emb_hbm.at[aidx_v.at[c]], bufs[c % NBUF], gsem)
        gaths[c].wait()
        base = wid * TOK_PER_W + c * CHUNK
        stores[c] = pltpu.async_copy(
            bufs[c % NBUF], app_hbm.at[pl.ds(base, CHUNK)], ssem)
    for c in range(NCHUNK - NBUF, NCHUNK):
        stores[c].wait()


def _run_k4b(emb_pad, aidx):
    mesh = plsc.VectorSubcoreMesh(core_axis_name="c", subcore_axis_name="s",
                                  num_cores=SC_NC, num_subcores=SC_NS)
    f = functools.partial(
        pl.kernel, mesh=mesh,
        out_type=jax.ShapeDtypeStruct((N, 128), jnp.float32),
        scratch_types=[
            pltpu.VMEM((NCHUNK, CHUNK), jnp.int32),
            pltpu.VMEM((CHUNK, 128), jnp.float32),
            pltpu.VMEM((CHUNK, 128), jnp.float32),
            pltpu.VMEM((CHUNK, 128), jnp.float32),
            pltpu.SemaphoreType.DMA,
            pltpu.SemaphoreType.DMA,
        ],
    )(_k4b_body)
    return f(emb_pad, aidx.reshape(NW * NCHUNK, CHUNK))


def _k5_body(y_ref, score_ref, x_ref, app_ref, w1_ref, b1_ref, w2_ref,
             b2_ref, wsig_ref, bsig_ref, wcol_ref, bcol_ref, out_ref):
    h = jnp.maximum(y_ref[...] * score_ref[...], 0.0)
    sigma = jnp.maximum(
        jnp.dot(h, wsig_ref[...], preferred_element_type=jnp.float32)
        + bsig_ref[...], 0.0)
    h1 = jnp.dot(h, w1_ref[...], preferred_element_type=jnp.float32)
    h1 = h1 + b1_ref[...]
    dirs = x_ref[:, 3:6]
    de = _posenc_cols(dirs, DIR_FREQ)
    feat = jnp.concatenate([h1, de, app_ref[...][:, 0:48]], axis=-1)
    h2 = jnp.maximum(
        jnp.dot(feat, w2_ref[...], preferred_element_type=jnp.float32)
        + b2_ref[...], 0.0)
    rgb = jax.nn.sigmoid(
        jnp.dot(h2, wcol_ref[...], preferred_element_type=jnp.float32)
        + bcol_ref[...])
    out_ref[...] = jnp.concatenate([rgb, sigma], axis=-1)


def _run_k5(y, score, x, app, W1, b1, W2, b2, Wsig, bsig, Wcol, bcol):
    full = lambda shape: pl.BlockSpec(shape, lambda i: (0,) * len(shape))
    return pl.pallas_call(
        _k5_body,
        grid=(NBLK,),
        in_specs=[
            pl.BlockSpec((TB, D), lambda i: (i, 0)),
            pl.BlockSpec((TB, 1), lambda i: (i, 0)),
            pl.BlockSpec((TB, 7), lambda i: (i, 0)),
            pl.BlockSpec((TB, 128), lambda i: (i, 0)),
            full((D, D)), full((D,)), full((331, 128)), full((128,)),
            full((D, 1)), full((1,)), full((128, 3)), full((3,)),
        ],
        out_specs=pl.BlockSpec((TB, 4), lambda i: (i, 0)),
        out_shape=jax.ShapeDtypeStruct((N, 4), jnp.float32),
        compiler_params=pltpu.CompilerParams(
            dimension_semantics=("arbitrary",)),
    )(y, score, x, app, W1, b1, W2, b2, Wsig, bsig, Wcol, bcol)


def kernel(x, W_xyz, b_xyz, W_eg1, b_eg1, W_eg2, b_eg2, ln_g, ln_b, Wg,
           We_pre, be_pre, We_skip, be_skip, We_post, be_post, W1, b1,
           W2, b2, Wsig, bsig, Wcol, bcol, emb_a):
    h, loc, cloc, score = _run_k1(x, W_xyz, b_xyz, W_eg1, b_eg1, W_eg2,
                                  b_eg2, ln_g, ln_b, Wg)
    buf = _run_k2(h, loc.reshape(N)).reshape(ECAP + 8, D)
    flat = _run_k3(buf[:ECAP], We_pre, be_pre, We_skip, be_skip,
                   We_post, be_post)
    aidx = x[:, 6].astype(jnp.int32)
    emb_pad = jnp.pad(emb_a, ((0, 0), (0, 128 - emb_a.shape[1])))
    y = _run_k4(flat, cloc.reshape(N)).reshape(N, D)
    app = _run_k4b(emb_pad, aidx)
    return _run_k5(y, score, x, app, W1, b1, W2, b2, Wsig, bsig, Wcol, bcol)


# speedup vs baseline: 1.3508x; 1.3508x over previous
import functools

import jax
import jax.numpy as jnp
import numpy as np
from jax import lax
from jax.experimental import pallas as pl
from jax.experimental.pallas import tpu as pltpu
from jax.experimental.pallas import tpu_sc as plsc

N = 32768
E = 64
D = 256
CAP = 640
ECAP = E * CAP
TB = 512
NBLK = N // TB
XYZ_FREQ = 12
DIR_FREQ = 4
REC = 128

SC_NC = 2
SC_NS = 16
NW = SC_NC * SC_NS
CHUNK = 128

XBUF_ROWS = ECAP + 256
TBL_ROWS = ECAP
BCH = 512


def _bounce(src, dst, vbuf, rows, src0, dst0):
    off = 0
    while rows > 0:
        take = min(BCH, rows)
        pltpu.sync_copy(src.at[pl.ds(src0 + off, take)],
                        vbuf.at[pl.ds(0, take)])
        pltpu.sync_copy(vbuf.at[pl.ds(0, take)],
                        dst.at[pl.ds(dst0 + off, take)])
        off += take
        rows -= take


def _posenc_perm(n_freqs):
    perm = list(range(3))
    for f in range(n_freqs):
        for c in range(3):
            perm.append(3 + 6 * f + c)
    for f in range(n_freqs):
        for c in range(3):
            perm.append(3 + 6 * f + 3 + c)
    return np.array(perm, dtype=np.int32)


def _posenc_packed(v, n_freqs):
    xf = jnp.concatenate([v * float(2.0 ** f) for f in range(n_freqs)],
                         axis=-1)
    return jnp.concatenate([v, jnp.sin(xf), jnp.cos(xf)], axis=-1)


def _k1_body(x_ref, wxyz_ref, bxyz_ref, weg1_ref, beg1_ref, weg2_ref,
             beg2_ref, lng_ref, lnb_ref, wg_ref,
             xs_ref, loc_ref, cloc_ref, cnt_ref):
    @pl.when(pl.program_id(0) == 0)
    def _():
        cnt_ref[...] = jnp.zeros((1, E), jnp.float32)

    xb = x_ref[...]
    xyz = xb[:, 0:3]
    pe = _posenc_packed(xyz, XYZ_FREQ)
    h = jnp.dot(pe, wxyz_ref[...], preferred_element_type=jnp.float32)
    h = h + bxyz_ref[...]

    eg = jnp.maximum(
        jnp.dot(h, weg1_ref[...], preferred_element_type=jnp.float32)
        + beg1_ref[...], 0.0)
    eg = jnp.dot(eg, weg2_ref[...], preferred_element_type=jnp.float32)
    eg = eg + beg2_ref[...]
    m = jnp.mean(eg, axis=-1, keepdims=True)
    v = jnp.mean((eg - m) ** 2, axis=-1, keepdims=True)
    gi = (eg - m) / jnp.sqrt(v + 1e-5) * lng_ref[...] + lnb_ref[...]

    logits = jnp.dot(gi, wg_ref[...], preferred_element_type=jnp.float32)
    lmax = jnp.max(logits, axis=-1, keepdims=True)
    score = 1.0 / jnp.sum(jnp.exp(logits - lmax), axis=-1, keepdims=True)

    eq = (logits == lmax).astype(jnp.float32)
    utri = (lax.broadcasted_iota(jnp.int32, (E, E), 0)
            <= lax.broadcasted_iota(jnp.int32, (E, E), 1)).astype(jnp.float32)
    csum1 = jnp.dot(eq, utri, preferred_element_type=jnp.float32)
    oh = eq * (csum1 == 1.0).astype(jnp.float32)

    ltri = (lax.broadcasted_iota(jnp.int32, (TB, TB), 0)
            >= lax.broadcasted_iota(jnp.int32, (TB, TB), 1)).astype(jnp.float32)
    csum0 = jnp.dot(ltri, oh, preferred_element_type=jnp.float32)
    pos_in = jnp.sum(csum0 * oh, axis=-1, keepdims=True) - 1.0

    cnt0 = cnt_ref[...]
    prev = lax.dot_general(oh, cnt0, (((1,), (1,)), ((), ())),
                           preferred_element_type=jnp.float32)
    pos = pos_in + prev
    cnt_ref[...] = cnt0 + jnp.sum(oh, axis=0, keepdims=True)

    col = lax.broadcasted_iota(jnp.int32, (TB, E), 1).astype(jnp.float32)
    idx_f = jnp.sum(oh * col, axis=-1, keepdims=True)
    keep = pos < float(CAP)
    loc_f = jnp.where(keep, idx_f * float(CAP) + pos, float(ECAP))
    loc_ref[...] = loc_f.astype(jnp.int32)
    cloc_ref[...] = jnp.minimum(loc_f.astype(jnp.int32), ECAP - 1)
    xs_ref[...] = jnp.concatenate(
        [xb, score, jnp.zeros((TB, REC - 8), jnp.float32)], axis=-1)


def _run_k1(x, W_xyz_p, b_xyz, W_eg1, b_eg1, W_eg2, b_eg2, ln_g, ln_b, Wg):
    full = lambda shape: pl.BlockSpec(shape, lambda i: (0,) * len(shape))
    return pl.pallas_call(
        _k1_body,
        grid=(NBLK,),
        in_specs=[
            pl.BlockSpec((TB, 7), lambda i: (i, 0)),
            full((75, D)), full((D,)), full((D, D)), full((D,)),
            full((D, D)), full((D,)), full((D,)), full((D,)), full((D, E)),
        ],
        out_specs=[
            pl.BlockSpec((TB, REC), lambda i: (i, 0)),
            pl.BlockSpec((TB, 1), lambda i: (i, 0)),
            pl.BlockSpec((TB, 1), lambda i: (i, 0)),
        ],
        out_shape=[
            jax.ShapeDtypeStruct((N, REC), jnp.float32),
            jax.ShapeDtypeStruct((N, 1), jnp.int32),
            jax.ShapeDtypeStruct((N, 1), jnp.int32),
        ],
        scratch_shapes=[pltpu.VMEM((1, E), jnp.float32)],
        compiler_params=pltpu.CompilerParams(
            dimension_semantics=("arbitrary",)),
    )(x, W_xyz_p, b_xyz, W_eg1, b_eg1, W_eg2, b_eg2, ln_g, ln_b, Wg)


NBUF = 3
NCHUNK = N // NW // CHUNK
TOK_PER_W = N // NW


def _k2_body(xs_hbm, loc_hbm, xbuf_hbm, idx_v, b0, b1, b2, lsem, ssem):
    wid = lax.axis_index("s") * SC_NC + lax.axis_index("c")
    bufs = [b0, b1, b2]
    pltpu.sync_copy(loc_hbm.at[pl.ds(wid * NCHUNK, NCHUNK)], idx_v)
    loads, scats = {}, {}
    for c in range(NBUF):
        base = wid * TOK_PER_W + c * CHUNK
        loads[c] = pltpu.async_copy(
            xs_hbm.at[pl.ds(base, CHUNK)], bufs[c], lsem)
    for c in range(NCHUNK):
        if c >= NBUF:
            scats[c - NBUF].wait()
            base = wid * TOK_PER_W + c * CHUNK
            loads[c] = pltpu.async_copy(
                xs_hbm.at[pl.ds(base, CHUNK)], bufs[c % NBUF], lsem)
        loads[c].wait()
        scats[c] = pltpu.async_copy(
            bufs[c % NBUF], xbuf_hbm.at[idx_v.at[c]], ssem)
    for c in range(NCHUNK - NBUF, NCHUNK):
        scats[c].wait()


def _run_k2(xs, loc):
    mesh = plsc.VectorSubcoreMesh(core_axis_name="c", subcore_axis_name="s",
                                  num_cores=SC_NC, num_subcores=SC_NS)
    f = functools.partial(
        pl.kernel, mesh=mesh,
        out_type=jax.ShapeDtypeStruct((XBUF_ROWS, REC), jnp.float32),
        scratch_types=[
            pltpu.VMEM((NCHUNK, CHUNK), jnp.int32),
            pltpu.VMEM((CHUNK, REC), jnp.float32),
            pltpu.VMEM((CHUNK, REC), jnp.float32),
            pltpu.VMEM((CHUNK, REC), jnp.float32),
            pltpu.SemaphoreType.DMA,
            pltpu.SemaphoreType.DMA,
        ],
    )(_k2_body)
    return f(xs, loc.reshape(N // CHUNK, CHUNK))


def _k3_body(xbuf_ref, wxyz_ref, bxyz_ref, wpre_ref, bpre_ref, wskip_ref,
             bskip_ref, wpost_ref, bpost_ref, w1_ref, b1_ref, w2_ref,
             b2_ref, wsig_ref, bsig_ref, wcol_ref, bcol_ref, emb_ref,
             out_ref):
    xb = xbuf_ref[...]
    xyz = xb[:, 0:3]
    pe = _posenc_packed(xyz, XYZ_FREQ)
    h = jnp.dot(pe, wxyz_ref[...], preferred_element_type=jnp.float32)
    h = h + bxyz_ref[...]

    hb = h
    for j in range(3):
        hb = jnp.maximum(
            jnp.dot(hb, wpre_ref[0, j], preferred_element_type=jnp.float32)
            + bpre_ref[0, j], 0.0)
    hb2 = jnp.concatenate([h, hb], axis=-1)
    hb = jnp.maximum(
        jnp.dot(hb2, wskip_ref[0], preferred_element_type=jnp.float32)
        + bskip_ref[0, 0], 0.0)
    for j in range(3):
        hb = jnp.maximum(
            jnp.dot(hb, wpost_ref[0, j], preferred_element_type=jnp.float32)
            + bpost_ref[0, j], 0.0)

    score = xb[:, 7:8]
    hp = jnp.maximum(hb * score, 0.0)
    sigma = jnp.maximum(
        jnp.dot(hp, wsig_ref[...], preferred_element_type=jnp.float32)
        + bsig_ref[...], 0.0)
    h1 = jnp.dot(hp, w1_ref[...], preferred_element_type=jnp.float32)
    h1 = h1 + b1_ref[...]
    de = _posenc_packed(xb[:, 3:6], DIR_FREQ)
    aid = xb[:, 6:7]
    acol = lax.broadcasted_iota(jnp.int32, (CAP, 1024), 1).astype(jnp.float32)
    aoh = (acol == aid).astype(jnp.float32)
    app = jnp.dot(aoh, emb_ref[...], preferred_element_type=jnp.float32)
    feat = jnp.concatenate([h1, de, app], axis=-1)
    h2 = jnp.maximum(
        jnp.dot(feat, w2_ref[...], preferred_element_type=jnp.float32)
        + b2_ref[...], 0.0)
    rgb = jax.nn.sigmoid(
        jnp.dot(h2, wcol_ref[...], preferred_element_type=jnp.float32)
        + bcol_ref[...])
    out_ref[...] = jnp.concatenate(
        [rgb, sigma, jnp.zeros((CAP, REC - 4), jnp.float32)], axis=-1)


def _run_k3(xbuf, W_xyz_p, b_xyz, We_pre, be_pre, We_skip, be_skip, We_post,
            be_post, W1, b1, W2_p, b2, Wsig, bsig, Wcol, bcol, emb_a):
    full = lambda shape: pl.BlockSpec(shape, lambda e: (0,) * len(shape))
    return pl.pallas_call(
        _k3_body,
        grid=(E,),
        in_specs=[
            pl.BlockSpec((CAP, REC), lambda e: (e, 0)),
            full((75, D)), full((D,)),
            pl.BlockSpec((1, 3, D, D), lambda e: (e, 0, 0, 0)),
            pl.BlockSpec((1, 3, D), lambda e: (e, 0, 0)),
            pl.BlockSpec((1, 2 * D, D), lambda e: (e, 0, 0)),
            pl.BlockSpec((1, 1, D), lambda e: (e, 0, 0)),
            pl.BlockSpec((1, 3, D, D), lambda e: (e, 0, 0, 0)),
            pl.BlockSpec((1, 3, D), lambda e: (e, 0, 0)),
            full((D, D)), full((D,)), full((331, 128)), full((128,)),
            full((D, 1)), full((1,)), full((128, 3)), full((3,)),
            full((1024, 48)),
        ],
        out_specs=pl.BlockSpec((CAP, REC), lambda e: (e, 0)),
        out_shape=jax.ShapeDtypeStruct((ECAP, REC), jnp.float32),
        compiler_params=pltpu.CompilerParams(
            dimension_semantics=("arbitrary",)),
    )(xbuf, W_xyz_p, b_xyz, We_pre, be_pre, We_skip,
      be_skip.reshape(E, 1, D), We_post, be_post, W1, b1, W2_p, b2,
      Wsig, bsig, Wcol, bcol, emb_a)


def _k5d_body(x_ref, b1_ref, w2_ref, b2_ref, bsig_ref, wcol_ref, bcol_ref,
              emb_ref, out_ref):
    xb = x_ref[...]
    de = _posenc_packed(xb[:, 3:6], DIR_FREQ)
    aid = xb[:, 6:7]
    acol = lax.broadcasted_iota(jnp.int32, (TB, 1024), 1).astype(jnp.float32)
    aoh = (acol == aid).astype(jnp.float32)
    app = jnp.dot(aoh, emb_ref[...], preferred_element_type=jnp.float32)
    h1 = jnp.zeros((TB, D), jnp.float32) + b1_ref[...]
    feat = jnp.concatenate([h1, de, app], axis=-1)
    h2 = jnp.maximum(
        jnp.dot(feat, w2_ref[...], preferred_element_type=jnp.float32)
        + b2_ref[...], 0.0)
    rgb = jax.nn.sigmoid(
        jnp.dot(h2, wcol_ref[...], preferred_element_type=jnp.float32)
        + bcol_ref[...])
    sigma = jnp.zeros((TB, 1), jnp.float32) + jnp.maximum(bsig_ref[...], 0.0)
    out_ref[...] = jnp.concatenate(
        [rgb, sigma, jnp.zeros((TB, REC - 4), jnp.float32)], axis=-1)


def _run_k5d(x, b1, W2_p, b2, bsig, Wcol, bcol, emb_a):
    full = lambda shape: pl.BlockSpec(shape, lambda i: (0,) * len(shape))
    return pl.pallas_call(
        _k5d_body,
        grid=(NBLK,),
        in_specs=[
            pl.BlockSpec((TB, 7), lambda i: (i, 0)),
            full((D,)), full((331, 128)), full((128,)), full((1,)),
            full((128, 3)), full((3,)), full((1024, 48)),
        ],
        out_specs=pl.BlockSpec((TB, REC), lambda i: (i, 0)),
        out_shape=jax.ShapeDtypeStruct((N, REC), jnp.float32),
        compiler_params=pltpu.CompilerParams(
            dimension_semantics=("arbitrary",)),
    )(x, b1, W2_p, b2, bsig, Wcol, bcol, emb_a)


def _k4_body(slots_hbm, cloc_hbm, o_hbm, idx_v, g0, g1, g2, gsem, ssem):
    wid = lax.axis_index("s") * SC_NC + lax.axis_index("c")
    gbufs = [g0, g1, g2]
    pltpu.sync_copy(cloc_hbm.at[pl.ds(wid * NCHUNK, NCHUNK)], idx_v)
    gaths, stores = {}, {}
    for c in range(NBUF):
        gaths[c] = pltpu.async_copy(
            slots_hbm.at[idx_v.at[c]], gbufs[c], gsem)
    for c in range(NCHUNK):
        if c >= NBUF:
            stores[c - NBUF].wait()
            gaths[c] = pltpu.async_copy(
                slots_hbm.at[idx_v.at[c]], gbufs[c % NBUF], gsem)
        gaths[c].wait()
        base = wid * TOK_PER_W + c * CHUNK
        stores[c] = pltpu.async_copy(
            gbufs[c % NBUF], o_hbm.at[pl.ds(base, CHUNK)], ssem)
    for c in range(NCHUNK - NBUF, NCHUNK):
        stores[c].wait()


def _run_k4(slots, cloc):
    mesh = plsc.VectorSubcoreMesh(core_axis_name="c", subcore_axis_name="s",
                                  num_cores=SC_NC, num_subcores=SC_NS)
    f = functools.partial(
        pl.kernel, mesh=mesh,
        out_type=jax.ShapeDtypeStruct((N, REC), jnp.float32),
        scratch_types=[
            pltpu.VMEM((NCHUNK, CHUNK), jnp.int32),
            pltpu.VMEM((CHUNK, REC), jnp.float32),
            pltpu.VMEM((CHUNK, REC), jnp.float32),
            pltpu.VMEM((CHUNK, REC), jnp.float32),
            pltpu.SemaphoreType.DMA,
            pltpu.SemaphoreType.DMA,
        ],
    )(_k4_body)
    return f(slots, cloc.reshape(N // CHUNK, CHUNK))


def _k6_body(o_ref, drop_ref, loc_ref, out_ref):
    keep = loc_ref[...] < ECAP
    out_ref[...] = jnp.where(keep, o_ref[...][:, 0:4],
                             drop_ref[...][:, 0:4])


def _run_k6(o8, drop, loc):
    return pl.pallas_call(
        _k6_body,
        grid=(NBLK,),
        in_specs=[
            pl.BlockSpec((TB, REC), lambda i: (i, 0)),
            pl.BlockSpec((TB, REC), lambda i: (i, 0)),
            pl.BlockSpec((TB, 1), lambda i: (i, 0)),
        ],
        out_specs=pl.BlockSpec((TB, 4), lambda i: (i, 0)),
        out_shape=jax.ShapeDtypeStruct((N, 4), jnp.float32),
        compiler_params=pltpu.CompilerParams(
            dimension_semantics=("arbitrary",)),
    )(o8, drop, loc)


def kernel(x, W_xyz, b_xyz, W_eg1, b_eg1, W_eg2, b_eg2, ln_g, ln_b, Wg,
           We_pre, be_pre, We_skip, be_skip, We_post, be_post, W1, b1,
           W2, b2, Wsig, bsig, Wcol, bcol, emb_a):
    W_xyz_p = W_xyz[_posenc_perm(XYZ_FREQ)]
    dperm = _posenc_perm(DIR_FREQ)
    W2_p = jnp.concatenate(
        [W2[0:256], W2[256:283][dperm], W2[283:331]], axis=0)
    xs, loc, cloc = _run_k1(x, W_xyz_p, b_xyz, W_eg1, b_eg1, W_eg2, b_eg2,
                            ln_g, ln_b, Wg)
    xbuf = _run_k2(xs, loc.reshape(N))
    slots = _run_k3(xbuf[:ECAP], W_xyz_p, b_xyz, We_pre, be_pre, We_skip,
                    be_skip, We_post, be_post, W1, b1, W2_p, b2, Wsig,
                    bsig, Wcol, bcol, emb_a)
    drop = _run_k5d(x, b1, W2_p, b2, bsig, Wcol, bcol, emb_a)
    o8 = slots[cloc.reshape(N)]
    return _run_k6(o8, drop, loc)
